# baseline (speedup 1.0000x reference)
import functools

import jax
import jax.numpy as jnp
from jax.experimental import pallas as pl
from jax.experimental.pallas import tpu as pltpu
from jax.experimental.pallas import tpu_sc as plsc

B, N, D, F = 2, 4096, 2, 512
K_SIM = 8
NB_ITER = 4
THR_SIM = 0.5
THR_OUT = 0.1
MAX_DIST = 0.2

TM = 256
NT = N // TM
TV = 1024
NV = N // TV
SL = TM // 128


def _dot(a, b, dims):
    return jax.lax.dot_general(a, b, (dims, ((), ())),
                               preferred_element_type=jnp.float32)


def _features_body(xy_ref, W1_ref, b1_ref, W2_ref, f_ref):
    xy = xy_ref[0]
    h = jnp.tanh(_dot(xy, W1_ref[...], ((1,), (0,))) + b1_ref[...][None, :])
    f = _dot(h, W2_ref[...], ((1,), (0,)))
    nrm = jnp.sqrt(jnp.sum(f * f, axis=-1, keepdims=True))
    f_ref[0] = f / (nrm + 1e-8)


def _features(xys, W1, b1, W2):
    return pl.pallas_call(
        _features_body,
        grid=(4,),
        in_specs=[
            pl.BlockSpec((1, N, D), lambda c: (c, 0, 0)),
            pl.BlockSpec((D, 128), lambda c: (0, 0)),
            pl.BlockSpec((128,), lambda c: (0,)),
            pl.BlockSpec((128, F), lambda c: (0, 0)),
        ],
        out_specs=pl.BlockSpec((1, N, F), lambda c: (c, 0, 0)),
        out_shape=jax.ShapeDtypeStruct((4, N, F), jnp.float32),
    )(xys, W1, b1, W2)


def _kbuild_body(eps_ref, f1_ref, f2_ref, xy1_ref, xy2_ref,
                 K_ref, rs_ref, dam_ref):
    eps = eps_ref[0]
    f1 = f1_ref[0]
    f2 = f2_ref[0]
    S = _dot(f1, f2, ((1,), (1,)))
    C = 1.0 - S
    x = xy1_ref[0]
    y = xy2_ref[0]
    xx = jnp.sum(x * x, axis=-1)
    yy = jnp.sum(y * y, axis=-1)
    cr = _dot(x, y, ((1,), (1,)))
    d12 = xx[:, None] + yy[None, :] - 2.0 * cr
    support = (d12 < MAX_DIST ** 2).astype(jnp.float32)
    Km = jnp.exp(-C / eps) * support
    K_ref[0] = Km
    rs_ref[0, 0] = jnp.sum(Km, axis=1, keepdims=True)
    dam_ref[0, 0] = jnp.argmin(d12, axis=1, keepdims=True).astype(jnp.int32)


def _kbuild(eps, f1, f2, xy1, xy2):
    return pl.pallas_call(
        _kbuild_body,
        grid=(B, NT),
        in_specs=[
            pl.BlockSpec(memory_space=pltpu.SMEM),
            pl.BlockSpec((1, TM, F), lambda b, i: (b, i, 0)),
            pl.BlockSpec((1, N, F), lambda b, i: (b, 0, 0)),
            pl.BlockSpec((1, TM, D), lambda b, i: (b, i, 0)),
            pl.BlockSpec((1, N, D), lambda b, i: (b, 0, 0)),
        ],
        out_specs=[
            pl.BlockSpec((1, TM, N), lambda b, i: (b, i, 0)),
            pl.BlockSpec((1, 1, TM, 1), lambda b, i: (b, i, 0, 0)),
            pl.BlockSpec((1, 1, TM, 1), lambda b, i: (b, i, 0, 0)),
        ],
        out_shape=[
            jax.ShapeDtypeStruct((B, N, N), jnp.float32),
            jax.ShapeDtypeStruct((B, NT, TM, 1), jnp.float32),
            jax.ShapeDtypeStruct((B, NT, TM, 1), jnp.int32),
        ],
    )(eps, f1, f2, xy1, xy2)


def _colmv_body(K_ref, a_ref, out_ref):
    i = pl.program_id(1)
    Km = K_ref[0]
    a = a_ref[0, 0]
    part = jnp.sum(Km * a, axis=0, keepdims=True)

    @pl.when(i == 0)
    def _():
        out_ref[0] = part

    @pl.when(i > 0)
    def _():
        out_ref[0] = out_ref[0] + part


def _colmv(K, a_t):
    return pl.pallas_call(
        _colmv_body,
        grid=(B, NV),
        in_specs=[
            pl.BlockSpec((1, TV, N), lambda b, i: (b, i, 0)),
            pl.BlockSpec((1, 1, TV, 1), lambda b, i: (b, i, 0, 0)),
        ],
        out_specs=pl.BlockSpec((1, 1, N), lambda b, i: (b, 0, 0)),
        out_shape=jax.ShapeDtypeStruct((B, 1, N), jnp.float32),
    )(K, a_t)


def _rowmv_body(K_ref, b_ref, out_ref):
    Km = K_ref[0]
    bv = b_ref[0]
    out_ref[0, 0] = jnp.sum(Km * bv, axis=1, keepdims=True)


def _rowmv(K, b_v):
    return pl.pallas_call(
        _rowmv_body,
        grid=(B, NV),
        in_specs=[
            pl.BlockSpec((1, TV, N), lambda b, i: (b, i, 0)),
            pl.BlockSpec((1, 1, N), lambda b, i: (b, 0, 0)),
        ],
        out_specs=pl.BlockSpec((1, 1, TV, 1), lambda b, i: (b, i, 0, 0)),
        out_shape=jax.ShapeDtypeStruct((B, NV, TV, 1), jnp.float32),
    )(K, b_v)


def _tpass_body(K_ref, a_ref, b_ref, ridx_ref, cmax_ref, cidx_ref):
    i = pl.program_id(1)
    Km = K_ref[0]
    a = a_ref[0, 0]
    bv = b_ref[0]
    T = (a * Km) * bv
    ridx_ref[0, 0] = jnp.argmax(T, axis=1, keepdims=True).astype(jnp.int32)
    cm = jnp.max(T, axis=0, keepdims=True)
    ca = (jnp.argmax(T, axis=0, keepdims=True) + i * TV).astype(jnp.int32)

    @pl.when(i == 0)
    def _():
        cmax_ref[0] = cm
        cidx_ref[0] = ca

    @pl.when(i > 0)
    def _():
        prev_m = cmax_ref[0]
        prev_i = cidx_ref[0]
        upd = cm > prev_m
        cmax_ref[0] = jnp.where(upd, cm, prev_m)
        cidx_ref[0] = jnp.where(upd, ca, prev_i)


def _tpass(K, a_t, b_v):
    return pl.pallas_call(
        _tpass_body,
        grid=(B, NV),
        in_specs=[
            pl.BlockSpec((1, TV, N), lambda b, i: (b, i, 0)),
            pl.BlockSpec((1, 1, TV, 1), lambda b, i: (b, i, 0, 0)),
            pl.BlockSpec((1, 1, N), lambda b, i: (b, 0, 0)),
        ],
        out_specs=[
            pl.BlockSpec((1, 1, TV, 1), lambda b, i: (b, i, 0, 0)),
            pl.BlockSpec((1, 1, N), lambda b, i: (b, 0, 0)),
            pl.BlockSpec((1, 1, N), lambda b, i: (b, 0, 0)),
        ],
        out_shape=[
            jax.ShapeDtypeStruct((B, NV, TV, 1), jnp.int32),
            jax.ShapeDtypeStruct((B, 1, N), jnp.float32),
            jax.ShapeDtypeStruct((B, 1, N), jnp.int32),
        ],
    )(K, a_t, b_v)


def _knn_body(xyt_ref, xya_ref, f_ref, nb_ref, desc_ref):
    x = xyt_ref[0]
    y = xya_ref[0]
    xx = jnp.sum(x * x, axis=-1)
    yy = jnp.sum(y * y, axis=-1)
    cr = _dot(x, y, ((1,), (1,)))
    d = xx[:, None] + yy[None, :] - 2.0 * cr
    col = jax.lax.broadcasted_iota(jnp.int32, (TM, N), 1)
    A = jnp.zeros((TM, N), jnp.float32)
    for k in range(K_SIM + 1):
        am = jnp.argmin(d, axis=1, keepdims=True).astype(jnp.int32)
        nb_ref[0, 0, :, k:k + 1] = am
        eq = col == am
        A = A + eq.astype(jnp.float32)
        d = jnp.where(eq, jnp.inf, d)
    desc_ref[0] = _dot(A, f_ref[0], ((1,), (0,))) / 9.0


def _knn(xys, fs):
    return pl.pallas_call(
        _knn_body,
        grid=(4, NT),
        in_specs=[
            pl.BlockSpec((1, TM, D), lambda c, i: (c, i, 0)),
            pl.BlockSpec((1, N, D), lambda c, i: (c, 0, 0)),
            pl.BlockSpec((1, N, F), lambda c, i: (c, 0, 0)),
        ],
        out_specs=[
            pl.BlockSpec((1, 1, TM, K_SIM + 1), lambda c, i: (c, i, 0, 0)),
            pl.BlockSpec((1, TM, F), lambda c, i: (c, i, 0)),
        ],
        out_shape=[
            jax.ShapeDtypeStruct((4, NT, TM, K_SIM + 1), jnp.int32),
            jax.ShapeDtypeStruct((4, N, F), jnp.float32),
        ],
    )(xys, xys, fs)


def _match_body(ri_ref, ci_ref, c0_ref, d1_ref, d2_ref, y0_ref, y1_ref,
                xy1_ref, out_ref, fl_ref):
    i = pl.program_id(1)
    ri = ri_ref[0, 0]
    ci = ci_ref[0]
    c0 = c0_ref[0, 0]
    col = jax.lax.broadcasted_iota(jnp.int32, (TM, N), 1)
    ohr = (col == ri).astype(jnp.int32)
    colg = jnp.sum(ohr * ci, axis=1, keepdims=True)
    row0 = jax.lax.broadcasted_iota(jnp.int32, (TM, 1), 0) + i * TM
    mutual = colg == row0
    idx_sub = jnp.where(mutual, ri, -1)
    idx_sub = jnp.where(c0 == idx_sub, idx_sub, -1)
    valid = idx_sub >= 0
    idxf = jnp.where(valid, idx_sub, 0)
    oh2 = (col == idxf).astype(jnp.float32)
    d2g = _dot(oh2, d2_ref[0], ((1,), (0,)))
    d1 = d1_ref[0]
    num = jnp.sum(d1 * d2g, axis=-1, keepdims=True)
    n1 = jnp.sqrt(jnp.sum(d1 * d1, axis=-1, keepdims=True))
    n2 = jnp.sqrt(jnp.sum(d2g * d2g, axis=-1, keepdims=True))
    cos = num / (n1 * n2 + 1e-8)
    idx_sub = jnp.where(valid & (cos > THR_SIM), idx_sub, -1)
    out_ref[0, 0] = idx_sub
    valid2 = idx_sub >= 0
    idxf2 = jnp.where(valid2, idx_sub, 0)
    oh3 = (col == idxf2).astype(jnp.float32)
    gx = jnp.sum(oh3 * y0_ref[0], axis=1, keepdims=True)
    gy = jnp.sum(oh3 * y1_ref[0], axis=1, keepdims=True)
    xt = xy1_ref[0]
    vf = valid2.astype(jnp.float32)
    fl_ref[0, 0] = jnp.concatenate(
        [gx - xt[:, 0:1], gy - xt[:, 1:2]], axis=1) * vf


def _match(ri_t, ci_v, c0_t, desc1, desc2, y0, y1, xy1):
    return pl.pallas_call(
        _match_body,
        grid=(B, NT),
        in_specs=[
            pl.BlockSpec((1, 1, TM, 1), lambda b, i: (b, i, 0, 0)),
            pl.BlockSpec((1, 1, N), lambda b, i: (b, 0, 0)),
            pl.BlockSpec((1, 1, TM, 1), lambda b, i: (b, i, 0, 0)),
            pl.BlockSpec((1, TM, F), lambda b, i: (b, i, 0)),
            pl.BlockSpec((1, N, F), lambda b, i: (b, 0, 0)),
            pl.BlockSpec((1, 1, N), lambda b, i: (b, 0, 0)),
            pl.BlockSpec((1, 1, N), lambda b, i: (b, 0, 0)),
            pl.BlockSpec((1, TM, D), lambda b, i: (b, i, 0)),
        ],
        out_specs=[
            pl.BlockSpec((1, 1, TM, 1), lambda b, i: (b, i, 0, 0)),
            pl.BlockSpec((1, 1, TM, 2), lambda b, i: (b, i, 0, 0)),
        ],
        out_shape=[
            jax.ShapeDtypeStruct((B, NT, TM, 1), jnp.int32),
            jax.ShapeDtypeStruct((B, NT, TM, 2), jnp.float32),
        ],
    )(ri_t, ci_v, c0_t, desc1, desc2, y0, y1, xy1)


_SC_CHUNK = N // 16


def _sc_nbmean_body(nb_ref, fx_ref, fy_ref, m_ref, mx_ref, my_ref,
                    nbv, tabx, taby, tabm, ox, oy):
    c = jax.lax.axis_index("c")
    s = jax.lax.axis_index("s")
    base = s * _SC_CHUNK
    pltpu.sync_copy(fx_ref.at[c], tabx)
    pltpu.sync_copy(fy_ref.at[c], taby)
    pltpu.sync_copy(m_ref.at[c], tabm)
    pltpu.sync_copy(nb_ref.at[c, :, pl.ds(base, _SC_CHUNK)], nbv)
    for g in range(_SC_CHUNK // 16):
        off = g * 16
        sx = jnp.zeros((16,), jnp.float32)
        sy = jnp.zeros((16,), jnp.float32)
        sm = jnp.zeros((16,), jnp.float32)
        for k in range(K_SIM + 1):
            idx = nbv[k, pl.ds(off, 16)]
            sx = sx + plsc.load_gather(tabx, [idx])
            sy = sy + plsc.load_gather(taby, [idx])
            sm = sm + plsc.load_gather(tabm, [idx])
        ox[pl.ds(off, 16)] = sx / (sm + 1e-8)
        oy[pl.ds(off, 16)] = sy / (sm + 1e-8)
    pltpu.sync_copy(ox, mx_ref.at[c, pl.ds(base, _SC_CHUNK)])
    pltpu.sync_copy(oy, my_ref.at[c, pl.ds(base, _SC_CHUNK)])


def _sc_nbmean(nb0T, fxm, fym, m):
    fn = functools.partial(
        pl.kernel,
        out_type=[
            jax.ShapeDtypeStruct((B, N), jnp.float32),
            jax.ShapeDtypeStruct((B, N), jnp.float32),
        ],
        mesh=plsc.VectorSubcoreMesh(core_axis_name="c", subcore_axis_name="s"),
        compiler_params=pltpu.CompilerParams(use_tc_tiling_on_sc=False,
                                             needs_layout_passes=False),
        scratch_types=[
            pltpu.VMEM((K_SIM + 1, _SC_CHUNK), jnp.int32),
            pltpu.VMEM((N,), jnp.float32),
            pltpu.VMEM((N,), jnp.float32),
            pltpu.VMEM((N,), jnp.float32),
            pltpu.VMEM((_SC_CHUNK,), jnp.float32),
            pltpu.VMEM((_SC_CHUNK,), jnp.float32),
        ],
    )(_sc_nbmean_body)
    return fn(nb0T, fxm, fym, m)


def _gather_b(x, idx):
    bidx = jnp.arange(x.shape[0]).reshape((-1,) + (1,) * (idx.ndim - 1))
    return x[bidx, idx]


def kernel(xy1, xy2, W1, b1, W2, eps_p, gamma_p):
    epsilon = jnp.exp(eps_p[0]) + 0.03
    gamma = jnp.exp(gamma_p[0])
    power = gamma / (gamma + epsilon)

    xys = jnp.concatenate([xy1, xy2], axis=0)
    fs = _features(xys, W1, b1, W2)
    f1, f2 = fs[:B], fs[B:]

    eps_arr = epsilon.reshape(1)
    K, rs, dam = _kbuild(eps_arr, f1, f2, xy1, xy2)

    prob = jnp.float32(1.0 / N)
    u = rs.reshape(B, NV, TV, 1) * prob
    a_t = (prob / (u + 1e-8)) ** power
    for it in range(NB_ITER):
        v = _colmv(K, a_t)
        b_v = (prob / (v + 1e-8)) ** power
        if it == NB_ITER - 1:
            break
        u = _rowmv(K, b_v)
        a_t = (prob / (u + 1e-8)) ** power

    ridx, _, cidx = _tpass(K, a_t, b_v)
    ridx = ridx.reshape(B, NT, TM, 1)

    nb4, descs = _knn(xys, fs)
    nb = nb4.reshape(4, N, K_SIM + 1)
    nb0 = nb[:B]
    desc1, desc2 = descs[:B], descs[B:]

    idx_sub2, fl_t = _match(ridx, cidx, dam, desc1, desc2,
                            xy2[..., 0].reshape(B, 1, N),
                            xy2[..., 1].reshape(B, 1, N), xy1)
    fl = fl_t.reshape(B, N, 2)
    valid2 = idx_sub2.reshape(B, N) >= 0
    m = valid2.astype(jnp.float32)

    nb0T = jnp.transpose(nb0, (0, 2, 1))
    mx, my = _sc_nbmean(nb0T, fl[..., 0] * m, fl[..., 1] * m, m)
    mean_nb = jnp.stack([mx, my], axis=-1)
    dev = jnp.linalg.norm(fl - mean_nb, axis=-1)
    idx_sub3 = jnp.where(valid2 & (dev < THR_OUT), idx_sub2.reshape(B, N), -1)

    track = idx_sub3 >= 0
    tm_ = track.astype(jnp.float32)
    flow = fl * tm_[..., None]
    gx, gy = _sc_nbmean(nb0T, flow[..., 0] * tm_, flow[..., 1] * tm_, tm_)
    flow_gri = jnp.stack([gx, gy], axis=-1)
    flow_gri = jnp.where(track[..., None], flow, flow_gri)
    return flow_gri


# speedup vs baseline: 1.0035x; 1.0035x over previous
import functools

import jax
import jax.numpy as jnp
from jax.experimental import pallas as pl
from jax.experimental.pallas import tpu as pltpu
from jax.experimental.pallas import tpu_sc as plsc

B, N, D, F = 2, 4096, 2, 512
K_SIM = 8
NB_ITER = 4
THR_SIM = 0.5
THR_OUT = 0.1
MAX_DIST = 0.2

TM = 256
NT = N // TM
TV = 512
NV = N // TV


def _dot(a, b, dims):
    return jax.lax.dot_general(a, b, (dims, ((), ())),
                               preferred_element_type=jnp.float32)


def _features_body(xy_ref, W1_ref, b1_ref, W2_ref, f_ref):
    xy = xy_ref[0]
    h = jnp.tanh(_dot(xy, W1_ref[...], ((1,), (0,))) + b1_ref[...][None, :])
    f = _dot(h, W2_ref[...], ((1,), (0,)))
    nrm = jnp.sqrt(jnp.sum(f * f, axis=-1, keepdims=True))
    f_ref[0] = f / (nrm + 1e-8)


def _features(xys, W1, b1, W2):
    return pl.pallas_call(
        _features_body,
        grid=(4,),
        in_specs=[
            pl.BlockSpec((1, N, D), lambda c: (c, 0, 0)),
            pl.BlockSpec((D, 128), lambda c: (0, 0)),
            pl.BlockSpec((128,), lambda c: (0,)),
            pl.BlockSpec((128, F), lambda c: (0, 0)),
        ],
        out_specs=pl.BlockSpec((1, N, F), lambda c: (c, 0, 0)),
        out_shape=jax.ShapeDtypeStruct((4, N, F), jnp.float32),
    )(xys, W1, b1, W2)


def _kbuild_body(eps_ref, f1_ref, f2_ref, xy1_ref, xy2_ref,
                 K_ref, rs_ref, dam_ref):
    eps = eps_ref[0]
    f1 = f1_ref[0]
    f2 = f2_ref[0]
    S = _dot(f1, f2, ((1,), (1,)))
    C = 1.0 - S
    x = xy1_ref[0]
    y = xy2_ref[0]
    xx = jnp.sum(x * x, axis=-1)
    yy = jnp.sum(y * y, axis=-1)
    cr = _dot(x, y, ((1,), (1,)))
    d12 = xx[:, None] + yy[None, :] - 2.0 * cr
    support = (d12 < MAX_DIST ** 2).astype(jnp.float32)
    Km = jnp.exp(-C / eps) * support
    K_ref[0] = Km
    rs_ref[0, 0] = jnp.sum(Km, axis=1, keepdims=True)
    dam_ref[0, 0] = jnp.argmin(d12, axis=1, keepdims=True).astype(jnp.int32)


def _kbuild(eps, f1, f2, xy1, xy2):
    return pl.pallas_call(
        _kbuild_body,
        grid=(B, NT),
        in_specs=[
            pl.BlockSpec(memory_space=pltpu.SMEM),
            pl.BlockSpec((1, TM, F), lambda b, i: (b, i, 0)),
            pl.BlockSpec((1, N, F), lambda b, i: (b, 0, 0)),
            pl.BlockSpec((1, TM, D), lambda b, i: (b, i, 0)),
            pl.BlockSpec((1, N, D), lambda b, i: (b, 0, 0)),
        ],
        out_specs=[
            pl.BlockSpec((1, TM, N), lambda b, i: (b, i, 0)),
            pl.BlockSpec((1, 1, TM, 1), lambda b, i: (b, i, 0, 0)),
            pl.BlockSpec((1, 1, TM, 1), lambda b, i: (b, i, 0, 0)),
        ],
        out_shape=[
            jax.ShapeDtypeStruct((B, N, N), jnp.float32),
            jax.ShapeDtypeStruct((B, NT, TM, 1), jnp.float32),
            jax.ShapeDtypeStruct((B, NT, TM, 1), jnp.int32),
        ],
    )(eps, f1, f2, xy1, xy2)


def _colmv_body(K_ref, a_ref, out_ref):
    i = pl.program_id(1)
    Km = K_ref[0]
    a = a_ref[0, 0]
    part = jnp.sum(Km * a, axis=0, keepdims=True)

    @pl.when(i == 0)
    def _():
        out_ref[0] = part

    @pl.when(i > 0)
    def _():
        out_ref[0] = out_ref[0] + part


def _colmv(K, a_t):
    return pl.pallas_call(
        _colmv_body,
        grid=(B, NV),
        in_specs=[
            pl.BlockSpec((1, TV, N), lambda b, i: (b, i, 0)),
            pl.BlockSpec((1, 1, TV, 1), lambda b, i: (b, i, 0, 0)),
        ],
        out_specs=pl.BlockSpec((1, 1, N), lambda b, i: (b, 0, 0)),
        out_shape=jax.ShapeDtypeStruct((B, 1, N), jnp.float32),
    )(K, a_t)


def _rowmv_body(K_ref, b_ref, out_ref):
    Km = K_ref[0]
    bv = b_ref[0]
    out_ref[0, 0] = jnp.sum(Km * bv, axis=1, keepdims=True)


def _rowmv(K, b_v):
    return pl.pallas_call(
        _rowmv_body,
        grid=(B, NV),
        in_specs=[
            pl.BlockSpec((1, TV, N), lambda b, i: (b, i, 0)),
            pl.BlockSpec((1, 1, N), lambda b, i: (b, 0, 0)),
        ],
        out_specs=pl.BlockSpec((1, 1, TV, 1), lambda b, i: (b, i, 0, 0)),
        out_shape=jax.ShapeDtypeStruct((B, NV, TV, 1), jnp.float32),
    )(K, b_v)


def _tpass_body(K_ref, a_ref, b_ref, ridx_ref, cmax_ref, cidx_ref):
    i = pl.program_id(1)
    Km = K_ref[0]
    a = a_ref[0, 0]
    bv = b_ref[0]
    T = (a * Km) * bv
    ridx_ref[0, 0] = jnp.argmax(T, axis=1, keepdims=True).astype(jnp.int32)
    cm = jnp.max(T, axis=0, keepdims=True)
    ca = (jnp.argmax(T, axis=0, keepdims=True) + i * TV).astype(jnp.int32)

    @pl.when(i == 0)
    def _():
        cmax_ref[0] = cm
        cidx_ref[0] = ca

    @pl.when(i > 0)
    def _():
        prev_m = cmax_ref[0]
        prev_i = cidx_ref[0]
        upd = cm > prev_m
        cmax_ref[0] = jnp.where(upd, cm, prev_m)
        cidx_ref[0] = jnp.where(upd, ca, prev_i)


def _tpass(K, a_t, b_v):
    return pl.pallas_call(
        _tpass_body,
        grid=(B, NV),
        in_specs=[
            pl.BlockSpec((1, TV, N), lambda b, i: (b, i, 0)),
            pl.BlockSpec((1, 1, TV, 1), lambda b, i: (b, i, 0, 0)),
            pl.BlockSpec((1, 1, N), lambda b, i: (b, 0, 0)),
        ],
        out_specs=[
            pl.BlockSpec((1, 1, TV, 1), lambda b, i: (b, i, 0, 0)),
            pl.BlockSpec((1, 1, N), lambda b, i: (b, 0, 0)),
            pl.BlockSpec((1, 1, N), lambda b, i: (b, 0, 0)),
        ],
        out_shape=[
            jax.ShapeDtypeStruct((B, NV, TV, 1), jnp.int32),
            jax.ShapeDtypeStruct((B, 1, N), jnp.float32),
            jax.ShapeDtypeStruct((B, 1, N), jnp.int32),
        ],
    )(K, a_t, b_v)


def _knn_body(xyt_ref, xya_ref, f_ref, nb_ref, desc_ref):
    x = xyt_ref[0]
    y = xya_ref[0]
    xx = jnp.sum(x * x, axis=-1)
    yy = jnp.sum(y * y, axis=-1)
    cr = _dot(x, y, ((1,), (1,)))
    d = xx[:, None] + yy[None, :] - 2.0 * cr
    col = jax.lax.broadcasted_iota(jnp.int32, (TM, N), 1)
    A = jnp.zeros((TM, N), jnp.float32)
    for k in range(K_SIM + 1):
        am = jnp.argmin(d, axis=1, keepdims=True).astype(jnp.int32)
        nb_ref[0, 0, :, k:k + 1] = am
        eq = col == am
        A = A + eq.astype(jnp.float32)
        d = jnp.where(eq, jnp.inf, d)
    desc_ref[0] = _dot(A, f_ref[0], ((1,), (0,))) / 9.0


def _knn(xys, fs):
    return pl.pallas_call(
        _knn_body,
        grid=(4, NT),
        in_specs=[
            pl.BlockSpec((1, TM, D), lambda c, i: (c, i, 0)),
            pl.BlockSpec((1, N, D), lambda c, i: (c, 0, 0)),
            pl.BlockSpec((1, N, F), lambda c, i: (c, 0, 0)),
        ],
        out_specs=[
            pl.BlockSpec((1, 1, TM, K_SIM + 1), lambda c, i: (c, i, 0, 0)),
            pl.BlockSpec((1, TM, F), lambda c, i: (c, i, 0)),
        ],
        out_shape=[
            jax.ShapeDtypeStruct((4, NT, TM, K_SIM + 1), jnp.int32),
            jax.ShapeDtypeStruct((4, N, F), jnp.float32),
        ],
    )(xys, xys, fs)


def _match_body(ri_ref, ci_ref, c0_ref, d1_ref, d2_ref, y0_ref, y1_ref,
                xy1_ref, out_ref, fl_ref):
    i = pl.program_id(1)
    ri = ri_ref[0, 0]
    ci = ci_ref[0]
    c0 = c0_ref[0, 0]
    col = jax.lax.broadcasted_iota(jnp.int32, (TM, N), 1)
    ohr = (col == ri).astype(jnp.int32)
    colg = jnp.sum(ohr * ci, axis=1, keepdims=True)
    row0 = jax.lax.broadcasted_iota(jnp.int32, (TM, 1), 0) + i * TM
    mutual = colg == row0
    idx_sub = jnp.where(mutual, ri, -1)
    idx_sub = jnp.where(c0 == idx_sub, idx_sub, -1)
    valid = idx_sub >= 0
    idxf = jnp.where(valid, idx_sub, 0)
    oh2 = (col == idxf).astype(jnp.float32)
    d2g = _dot(oh2, d2_ref[0], ((1,), (0,)))
    d1 = d1_ref[0]
    num = jnp.sum(d1 * d2g, axis=-1, keepdims=True)
    n1 = jnp.sqrt(jnp.sum(d1 * d1, axis=-1, keepdims=True))
    n2 = jnp.sqrt(jnp.sum(d2g * d2g, axis=-1, keepdims=True))
    cos = num / (n1 * n2 + 1e-8)
    idx_sub = jnp.where(valid & (cos > THR_SIM), idx_sub, -1)
    out_ref[0, 0] = idx_sub
    valid2 = idx_sub >= 0
    idxf2 = jnp.where(valid2, idx_sub, 0)
    oh3 = (col == idxf2).astype(jnp.float32)
    gx = jnp.sum(oh3 * y0_ref[0], axis=1, keepdims=True)
    gy = jnp.sum(oh3 * y1_ref[0], axis=1, keepdims=True)
    xt = xy1_ref[0]
    vf = valid2.astype(jnp.float32)
    fl_ref[0, 0] = jnp.concatenate(
        [gx - xt[:, 0:1], gy - xt[:, 1:2]], axis=1) * vf


def _match(ri_t, ci_v, c0_t, desc1, desc2, y0, y1, xy1):
    return pl.pallas_call(
        _match_body,
        grid=(B, NT),
        in_specs=[
            pl.BlockSpec((1, 1, TM, 1), lambda b, i: (b, i, 0, 0)),
            pl.BlockSpec((1, 1, N), lambda b, i: (b, 0, 0)),
            pl.BlockSpec((1, 1, TM, 1), lambda b, i: (b, i, 0, 0)),
            pl.BlockSpec((1, TM, F), lambda b, i: (b, i, 0)),
            pl.BlockSpec((1, N, F), lambda b, i: (b, 0, 0)),
            pl.BlockSpec((1, 1, N), lambda b, i: (b, 0, 0)),
            pl.BlockSpec((1, 1, N), lambda b, i: (b, 0, 0)),
            pl.BlockSpec((1, TM, D), lambda b, i: (b, i, 0)),
        ],
        out_specs=[
            pl.BlockSpec((1, 1, TM, 1), lambda b, i: (b, i, 0, 0)),
            pl.BlockSpec((1, 1, TM, 2), lambda b, i: (b, i, 0, 0)),
        ],
        out_shape=[
            jax.ShapeDtypeStruct((B, NT, TM, 1), jnp.int32),
            jax.ShapeDtypeStruct((B, NT, TM, 2), jnp.float32),
        ],
    )(ri_t, ci_v, c0_t, desc1, desc2, y0, y1, xy1)


_SC_CHUNK = N // 16


def _sc_nbmean_body(nb_ref, fx_ref, fy_ref, m_ref, mx_ref, my_ref,
                    nbv, tabx, taby, tabm, ox, oy):
    c = jax.lax.axis_index("c")
    s = jax.lax.axis_index("s")
    base = s * _SC_CHUNK
    pltpu.sync_copy(fx_ref.at[c], tabx)
    pltpu.sync_copy(fy_ref.at[c], taby)
    pltpu.sync_copy(m_ref.at[c], tabm)
    pltpu.sync_copy(nb_ref.at[c, :, pl.ds(base, _SC_CHUNK)], nbv)
    for g in range(_SC_CHUNK // 16):
        off = g * 16
        sx = jnp.zeros((16,), jnp.float32)
        sy = jnp.zeros((16,), jnp.float32)
        sm = jnp.zeros((16,), jnp.float32)
        for k in range(K_SIM + 1):
            idx = nbv[k, pl.ds(off, 16)]
            sx = sx + plsc.load_gather(tabx, [idx])
            sy = sy + plsc.load_gather(taby, [idx])
            sm = sm + plsc.load_gather(tabm, [idx])
        ox[pl.ds(off, 16)] = sx / (sm + 1e-8)
        oy[pl.ds(off, 16)] = sy / (sm + 1e-8)
    pltpu.sync_copy(ox, mx_ref.at[c, pl.ds(base, _SC_CHUNK)])
    pltpu.sync_copy(oy, my_ref.at[c, pl.ds(base, _SC_CHUNK)])


def _sc_nbmean(nb0T, fxm, fym, m):
    fn = functools.partial(
        pl.kernel,
        out_type=[
            jax.ShapeDtypeStruct((B, N), jnp.float32),
            jax.ShapeDtypeStruct((B, N), jnp.float32),
        ],
        mesh=plsc.VectorSubcoreMesh(core_axis_name="c", subcore_axis_name="s"),
        compiler_params=pltpu.CompilerParams(use_tc_tiling_on_sc=False,
                                             needs_layout_passes=False),
        scratch_types=[
            pltpu.VMEM((K_SIM + 1, _SC_CHUNK), jnp.int32),
            pltpu.VMEM((N,), jnp.float32),
            pltpu.VMEM((N,), jnp.float32),
            pltpu.VMEM((N,), jnp.float32),
            pltpu.VMEM((_SC_CHUNK,), jnp.float32),
            pltpu.VMEM((_SC_CHUNK,), jnp.float32),
        ],
    )(_sc_nbmean_body)
    return fn(nb0T, fxm, fym, m)


def kernel(xy1, xy2, W1, b1, W2, eps_p, gamma_p):
    epsilon = jnp.exp(eps_p[0]) + 0.03
    gamma = jnp.exp(gamma_p[0])
    power = gamma / (gamma + epsilon)

    xys = jnp.concatenate([xy1, xy2], axis=0)
    fs = _features(xys, W1, b1, W2)
    f1, f2 = fs[:B], fs[B:]

    eps_arr = epsilon.reshape(1)
    K, rs, dam = _kbuild(eps_arr, f1, f2, xy1, xy2)

    prob = jnp.float32(1.0 / N)
    u = rs.reshape(B, NV, TV, 1) * prob
    a_t = (prob / (u + 1e-8)) ** power
    for it in range(NB_ITER):
        v = _colmv(K, a_t)
        b_v = (prob / (v + 1e-8)) ** power
        if it == NB_ITER - 1:
            break
        u = _rowmv(K, b_v)
        a_t = (prob / (u + 1e-8)) ** power

    ridx, _, cidx = _tpass(K, a_t, b_v)
    ridx = ridx.reshape(B, NT, TM, 1)

    nb4, descs = _knn(xys, fs)
    nb = nb4.reshape(4, N, K_SIM + 1)
    nb0 = nb[:B]
    desc1, desc2 = descs[:B], descs[B:]

    idx_sub2, fl_t = _match(ridx, cidx, dam, desc1, desc2,
                            xy2[..., 0].reshape(B, 1, N),
                            xy2[..., 1].reshape(B, 1, N), xy1)
    fl = fl_t.reshape(B, N, 2)
    valid2 = idx_sub2.reshape(B, N) >= 0
    m = valid2.astype(jnp.float32)

    nb0T = jnp.transpose(nb0, (0, 2, 1))
    mx, my = _sc_nbmean(nb0T, fl[..., 0] * m, fl[..., 1] * m, m)
    mean_nb = jnp.stack([mx, my], axis=-1)
    dev = jnp.linalg.norm(fl - mean_nb, axis=-1)
    idx_sub3 = jnp.where(valid2 & (dev < THR_OUT), idx_sub2.reshape(B, N), -1)

    track = idx_sub3 >= 0
    tm_ = track.astype(jnp.float32)
    flow = fl * tm_[..., None]
    gx, gy = _sc_nbmean(nb0T, flow[..., 0] * tm_, flow[..., 1] * tm_, tm_)
    flow_gri = jnp.stack([gx, gy], axis=-1)
    flow_gri = jnp.where(track[..., None], flow, flow_gri)
    return flow_gri


# speedup vs baseline: 1.0066x; 1.0031x over previous
import functools

import jax
import jax.numpy as jnp
from jax.experimental import pallas as pl
from jax.experimental.pallas import tpu as pltpu
from jax.experimental.pallas import tpu_sc as plsc

B, N, D, F = 2, 4096, 2, 512
K_SIM = 8
NB_ITER = 4
THR_SIM = 0.5
THR_OUT = 0.1
MAX_DIST = 0.2

TM = 256
NT = N // TM
TV = 512
NV = N // TV


def _dot(a, b, dims):
    return jax.lax.dot_general(a, b, (dims, ((), ())),
                               preferred_element_type=jnp.float32)


def _features_body(xy_ref, W1_ref, b1_ref, W2_ref, f_ref):
    xy = xy_ref[0]
    h = jnp.tanh(_dot(xy, W1_ref[...], ((1,), (0,))) + b1_ref[...][None, :])
    f = _dot(h, W2_ref[...], ((1,), (0,)))
    nrm = jnp.sqrt(jnp.sum(f * f, axis=-1, keepdims=True))
    f_ref[0] = f / (nrm + 1e-8)


def _features(xys, W1, b1, W2):
    return pl.pallas_call(
        _features_body,
        grid=(4,),
        in_specs=[
            pl.BlockSpec((1, N, D), lambda c: (c, 0, 0)),
            pl.BlockSpec((D, 128), lambda c: (0, 0)),
            pl.BlockSpec((128,), lambda c: (0,)),
            pl.BlockSpec((128, F), lambda c: (0, 0)),
        ],
        out_specs=pl.BlockSpec((1, N, F), lambda c: (c, 0, 0)),
        out_shape=jax.ShapeDtypeStruct((4, N, F), jnp.float32),
    )(xys, W1, b1, W2)


def _kbuild_body(eps_ref, f1_ref, f2_ref, xy1_ref, xy2_ref,
                 K_ref, rs_ref, dam_ref):
    eps = eps_ref[0]
    f1 = f1_ref[0]
    f2 = f2_ref[0]
    S = _dot(f1, f2, ((1,), (1,)))
    C = 1.0 - S
    x = xy1_ref[0]
    y = xy2_ref[0]
    xx = jnp.sum(x * x, axis=-1)
    yy = jnp.sum(y * y, axis=-1)
    cr = _dot(x, y, ((1,), (1,)))
    d12 = xx[:, None] + yy[None, :] - 2.0 * cr
    support = (d12 < MAX_DIST ** 2).astype(jnp.float32)
    Km = jnp.exp(-C / eps) * support
    K_ref[0] = Km
    rs_ref[0, 0] = jnp.sum(Km, axis=1, keepdims=True)
    dam_ref[0, 0] = jnp.argmin(d12, axis=1, keepdims=True).astype(jnp.int32)


def _kbuild(eps, f1, f2, xy1, xy2):
    return pl.pallas_call(
        _kbuild_body,
        grid=(B, NV),
        in_specs=[
            pl.BlockSpec(memory_space=pltpu.SMEM),
            pl.BlockSpec((1, TV, F), lambda b, i: (b, i, 0)),
            pl.BlockSpec((1, N, F), lambda b, i: (b, 0, 0)),
            pl.BlockSpec((1, TV, D), lambda b, i: (b, i, 0)),
            pl.BlockSpec((1, N, D), lambda b, i: (b, 0, 0)),
        ],
        out_specs=[
            pl.BlockSpec((1, TV, N), lambda b, i: (b, i, 0)),
            pl.BlockSpec((1, 1, TV, 1), lambda b, i: (b, i, 0, 0)),
            pl.BlockSpec((1, 1, TV, 1), lambda b, i: (b, i, 0, 0)),
        ],
        out_shape=[
            jax.ShapeDtypeStruct((B, N, N), jnp.float32),
            jax.ShapeDtypeStruct((B, NV, TV, 1), jnp.float32),
            jax.ShapeDtypeStruct((B, NV, TV, 1), jnp.int32),
        ],
    )(eps, f1, f2, xy1, xy2)


def _colmv_body(K_ref, a_ref, out_ref):
    i = pl.program_id(1)
    Km = K_ref[0]
    a = a_ref[0, 0]
    part = jnp.sum(Km * a, axis=0, keepdims=True)

    @pl.when(i == 0)
    def _():
        out_ref[0] = part

    @pl.when(i > 0)
    def _():
        out_ref[0] = out_ref[0] + part


def _colmv(K, a_t):
    return pl.pallas_call(
        _colmv_body,
        grid=(B, NV),
        in_specs=[
            pl.BlockSpec((1, TV, N), lambda b, i: (b, i, 0)),
            pl.BlockSpec((1, 1, TV, 1), lambda b, i: (b, i, 0, 0)),
        ],
        out_specs=pl.BlockSpec((1, 1, N), lambda b, i: (b, 0, 0)),
        out_shape=jax.ShapeDtypeStruct((B, 1, N), jnp.float32),
    )(K, a_t)


def _rowmv_body(K_ref, b_ref, out_ref):
    Km = K_ref[0]
    bv = b_ref[0]
    out_ref[0, 0] = jnp.sum(Km * bv, axis=1, keepdims=True)


def _rowmv(K, b_v):
    return pl.pallas_call(
        _rowmv_body,
        grid=(B, NV),
        in_specs=[
            pl.BlockSpec((1, TV, N), lambda b, i: (b, i, 0)),
            pl.BlockSpec((1, 1, N), lambda b, i: (b, 0, 0)),
        ],
        out_specs=pl.BlockSpec((1, 1, TV, 1), lambda b, i: (b, i, 0, 0)),
        out_shape=jax.ShapeDtypeStruct((B, NV, TV, 1), jnp.float32),
    )(K, b_v)


def _tpass_body(K_ref, a_ref, b_ref, ridx_ref, cmax_ref, cidx_ref):
    i = pl.program_id(1)
    Km = K_ref[0]
    a = a_ref[0, 0]
    bv = b_ref[0]
    T = (a * Km) * bv
    ridx_ref[0, 0] = jnp.argmax(T, axis=1, keepdims=True).astype(jnp.int32)
    cm = jnp.max(T, axis=0, keepdims=True)
    ca = (jnp.argmax(T, axis=0, keepdims=True) + i * TV).astype(jnp.int32)

    @pl.when(i == 0)
    def _():
        cmax_ref[0] = cm
        cidx_ref[0] = ca

    @pl.when(i > 0)
    def _():
        prev_m = cmax_ref[0]
        prev_i = cidx_ref[0]
        upd = cm > prev_m
        cmax_ref[0] = jnp.where(upd, cm, prev_m)
        cidx_ref[0] = jnp.where(upd, ca, prev_i)


def _tpass(K, a_t, b_v):
    return pl.pallas_call(
        _tpass_body,
        grid=(B, NV),
        in_specs=[
            pl.BlockSpec((1, TV, N), lambda b, i: (b, i, 0)),
            pl.BlockSpec((1, 1, TV, 1), lambda b, i: (b, i, 0, 0)),
            pl.BlockSpec((1, 1, N), lambda b, i: (b, 0, 0)),
        ],
        out_specs=[
            pl.BlockSpec((1, 1, TV, 1), lambda b, i: (b, i, 0, 0)),
            pl.BlockSpec((1, 1, N), lambda b, i: (b, 0, 0)),
            pl.BlockSpec((1, 1, N), lambda b, i: (b, 0, 0)),
        ],
        out_shape=[
            jax.ShapeDtypeStruct((B, NV, TV, 1), jnp.int32),
            jax.ShapeDtypeStruct((B, 1, N), jnp.float32),
            jax.ShapeDtypeStruct((B, 1, N), jnp.int32),
        ],
    )(K, a_t, b_v)


def _knn_body(xyt_ref, xya_ref, f_ref, nb_ref, desc_ref):
    x = xyt_ref[0]
    y = xya_ref[0]
    xx = jnp.sum(x * x, axis=-1)
    yy = jnp.sum(y * y, axis=-1)
    cr = _dot(x, y, ((1,), (1,)))
    d = xx[:, None] + yy[None, :] - 2.0 * cr
    col = jax.lax.broadcasted_iota(jnp.int32, (TM, N), 1)
    A = jnp.zeros((TM, N), jnp.float32)
    for k in range(K_SIM + 1):
        am = jnp.argmin(d, axis=1, keepdims=True).astype(jnp.int32)
        nb_ref[0, 0, :, k:k + 1] = am
        eq = col == am
        A = A + eq.astype(jnp.float32)
        d = jnp.where(eq, jnp.inf, d)
    desc_ref[0] = _dot(A, f_ref[0], ((1,), (0,))) / 9.0


def _knn(xys, fs):
    return pl.pallas_call(
        _knn_body,
        grid=(4, NT),
        in_specs=[
            pl.BlockSpec((1, TM, D), lambda c, i: (c, i, 0)),
            pl.BlockSpec((1, N, D), lambda c, i: (c, 0, 0)),
            pl.BlockSpec((1, N, F), lambda c, i: (c, 0, 0)),
        ],
        out_specs=[
            pl.BlockSpec((1, 1, TM, K_SIM + 1), lambda c, i: (c, i, 0, 0)),
            pl.BlockSpec((1, TM, F), lambda c, i: (c, i, 0)),
        ],
        out_shape=[
            jax.ShapeDtypeStruct((4, NT, TM, K_SIM + 1), jnp.int32),
            jax.ShapeDtypeStruct((4, N, F), jnp.float32),
        ],
    )(xys, xys, fs)


def _match_body(ri_ref, ci_ref, c0_ref, d1_ref, d2_ref, y0_ref, y1_ref,
                xy1_ref, out_ref, fl_ref):
    i = pl.program_id(1)
    ri = ri_ref[0, 0]
    ci = ci_ref[0]
    c0 = c0_ref[0, 0]
    col = jax.lax.broadcasted_iota(jnp.int32, (TM, N), 1)
    ohr = (col == ri).astype(jnp.int32)
    colg = jnp.sum(ohr * ci, axis=1, keepdims=True)
    row0 = jax.lax.broadcasted_iota(jnp.int32, (TM, 1), 0) + i * TM
    mutual = colg == row0
    idx_sub = jnp.where(mutual, ri, -1)
    idx_sub = jnp.where(c0 == idx_sub, idx_sub, -1)
    valid = idx_sub >= 0
    idxf = jnp.where(valid, idx_sub, 0)
    oh2 = (col == idxf).astype(jnp.float32)
    d2g = _dot(oh2, d2_ref[0], ((1,), (0,)))
    d1 = d1_ref[0]
    num = jnp.sum(d1 * d2g, axis=-1, keepdims=True)
    n1 = jnp.sqrt(jnp.sum(d1 * d1, axis=-1, keepdims=True))
    n2 = jnp.sqrt(jnp.sum(d2g * d2g, axis=-1, keepdims=True))
    cos = num / (n1 * n2 + 1e-8)
    idx_sub = jnp.where(valid & (cos > THR_SIM), idx_sub, -1)
    out_ref[0, 0] = idx_sub
    valid2 = idx_sub >= 0
    idxf2 = jnp.where(valid2, idx_sub, 0)
    oh3 = (col == idxf2).astype(jnp.float32)
    gx = jnp.sum(oh3 * y0_ref[0], axis=1, keepdims=True)
    gy = jnp.sum(oh3 * y1_ref[0], axis=1, keepdims=True)
    xt = xy1_ref[0]
    vf = valid2.astype(jnp.float32)
    fl_ref[0, 0] = jnp.concatenate(
        [gx - xt[:, 0:1], gy - xt[:, 1:2]], axis=1) * vf


def _match(ri_t, ci_v, c0_t, desc1, desc2, y0, y1, xy1):
    return pl.pallas_call(
        _match_body,
        grid=(B, NT),
        in_specs=[
            pl.BlockSpec((1, 1, TM, 1), lambda b, i: (b, i, 0, 0)),
            pl.BlockSpec((1, 1, N), lambda b, i: (b, 0, 0)),
            pl.BlockSpec((1, 1, TM, 1), lambda b, i: (b, i, 0, 0)),
            pl.BlockSpec((1, TM, F), lambda b, i: (b, i, 0)),
            pl.BlockSpec((1, N, F), lambda b, i: (b, 0, 0)),
            pl.BlockSpec((1, 1, N), lambda b, i: (b, 0, 0)),
            pl.BlockSpec((1, 1, N), lambda b, i: (b, 0, 0)),
            pl.BlockSpec((1, TM, D), lambda b, i: (b, i, 0)),
        ],
        out_specs=[
            pl.BlockSpec((1, 1, TM, 1), lambda b, i: (b, i, 0, 0)),
            pl.BlockSpec((1, 1, TM, 2), lambda b, i: (b, i, 0, 0)),
        ],
        out_shape=[
            jax.ShapeDtypeStruct((B, NT, TM, 1), jnp.int32),
            jax.ShapeDtypeStruct((B, NT, TM, 2), jnp.float32),
        ],
    )(ri_t, ci_v, c0_t, desc1, desc2, y0, y1, xy1)


_SC_CHUNK = N // 16


def _sc_nbmean_body(nb_ref, fx_ref, fy_ref, m_ref, mx_ref, my_ref,
                    nbv, tabx, taby, tabm, ox, oy):
    c = jax.lax.axis_index("c")
    s = jax.lax.axis_index("s")
    base = s * _SC_CHUNK
    pltpu.sync_copy(fx_ref.at[c], tabx)
    pltpu.sync_copy(fy_ref.at[c], taby)
    pltpu.sync_copy(m_ref.at[c], tabm)
    pltpu.sync_copy(nb_ref.at[c, :, pl.ds(base, _SC_CHUNK)], nbv)
    for g in range(_SC_CHUNK // 16):
        off = g * 16
        sx = jnp.zeros((16,), jnp.float32)
        sy = jnp.zeros((16,), jnp.float32)
        sm = jnp.zeros((16,), jnp.float32)
        for k in range(K_SIM + 1):
            idx = nbv[k, pl.ds(off, 16)]
            sx = sx + plsc.load_gather(tabx, [idx])
            sy = sy + plsc.load_gather(taby, [idx])
            sm = sm + plsc.load_gather(tabm, [idx])
        ox[pl.ds(off, 16)] = sx / (sm + 1e-8)
        oy[pl.ds(off, 16)] = sy / (sm + 1e-8)
    pltpu.sync_copy(ox, mx_ref.at[c, pl.ds(base, _SC_CHUNK)])
    pltpu.sync_copy(oy, my_ref.at[c, pl.ds(base, _SC_CHUNK)])


def _sc_nbmean(nb0T, fxm, fym, m):
    fn = functools.partial(
        pl.kernel,
        out_type=[
            jax.ShapeDtypeStruct((B, N), jnp.float32),
            jax.ShapeDtypeStruct((B, N), jnp.float32),
        ],
        mesh=plsc.VectorSubcoreMesh(core_axis_name="c", subcore_axis_name="s"),
        compiler_params=pltpu.CompilerParams(use_tc_tiling_on_sc=False,
                                             needs_layout_passes=False),
        scratch_types=[
            pltpu.VMEM((K_SIM + 1, _SC_CHUNK), jnp.int32),
            pltpu.VMEM((N,), jnp.float32),
            pltpu.VMEM((N,), jnp.float32),
            pltpu.VMEM((N,), jnp.float32),
            pltpu.VMEM((_SC_CHUNK,), jnp.float32),
            pltpu.VMEM((_SC_CHUNK,), jnp.float32),
        ],
    )(_sc_nbmean_body)
    return fn(nb0T, fxm, fym, m)


def kernel(xy1, xy2, W1, b1, W2, eps_p, gamma_p):
    epsilon = jnp.exp(eps_p[0]) + 0.03
    gamma = jnp.exp(gamma_p[0])
    power = gamma / (gamma + epsilon)

    xys = jnp.concatenate([xy1, xy2], axis=0)
    fs = _features(xys, W1, b1, W2)
    f1, f2 = fs[:B], fs[B:]

    eps_arr = epsilon.reshape(1)
    K, rs, dam = _kbuild(eps_arr, f1, f2, xy1, xy2)

    prob = jnp.float32(1.0 / N)
    u = rs.reshape(B, NV, TV, 1) * prob
    a_t = (prob / (u + 1e-8)) ** power
    for it in range(NB_ITER):
        v = _colmv(K, a_t)
        b_v = (prob / (v + 1e-8)) ** power
        if it == NB_ITER - 1:
            break
        u = _rowmv(K, b_v)
        a_t = (prob / (u + 1e-8)) ** power

    ridx, _, cidx = _tpass(K, a_t, b_v)
    ridx = ridx.reshape(B, NT, TM, 1)

    nb4, descs = _knn(xys, fs)
    nb = nb4.reshape(4, N, K_SIM + 1)
    nb0 = nb[:B]
    desc1, desc2 = descs[:B], descs[B:]

    idx_sub2, fl_t = _match(ridx, cidx, dam.reshape(B, NT, TM, 1), desc1, desc2,
                            xy2[..., 0].reshape(B, 1, N),
                            xy2[..., 1].reshape(B, 1, N), xy1)
    fl = fl_t.reshape(B, N, 2)
    valid2 = idx_sub2.reshape(B, N) >= 0
    m = valid2.astype(jnp.float32)

    nb0T = jnp.transpose(nb0, (0, 2, 1))
    mx, my = _sc_nbmean(nb0T, fl[..., 0] * m, fl[..., 1] * m, m)
    mean_nb = jnp.stack([mx, my], axis=-1)
    dev = jnp.linalg.norm(fl - mean_nb, axis=-1)
    idx_sub3 = jnp.where(valid2 & (dev < THR_OUT), idx_sub2.reshape(B, N), -1)

    track = idx_sub3 >= 0
    tm_ = track.astype(jnp.float32)
    flow = fl * tm_[..., None]
    gx, gy = _sc_nbmean(nb0T, flow[..., 0] * tm_, flow[..., 1] * tm_, tm_)
    flow_gri = jnp.stack([gx, gy], axis=-1)
    flow_gri = jnp.where(track[..., None], flow, flow_gri)
    return flow_gri
